# baseline (speedup 1.0000x reference)
import jax
import jax.numpy as jnp
import numpy as np
from jax.experimental import pallas as pl
from jax.sharding import Mesh, PartitionSpec as P

B, K, D = 8192, 8192, 64
BB = 512
KC = 2048


def _vq_kernel(x_ref, cb_ref, xsq_ref, csq_ref, out_ref):
    x = x_ref[...]
    x_sq = xsq_ref[...]

    def body(kc, carry):
        run_min, run_arg = carry
        c = cb_ref[pl.ds(kc * KC, KC), :]
        c_sq = csq_ref[:, pl.ds(kc * KC, KC)]
        cross = jax.lax.dot_general(
            x, c, (((1,), (1,)), ((), ())),
            preferred_element_type=jnp.float32)
        dists = x_sq + c_sq - 2.0 * cross
        loc_min = jnp.min(dists, axis=1, keepdims=True)
        idx = jax.lax.broadcasted_iota(jnp.int32, (BB, KC), 1) + kc * KC
        loc_arg = jnp.min(jnp.where(dists == loc_min, idx, K),
                          axis=1, keepdims=True)
        better = loc_min < run_min
        new_min = jnp.where(better, loc_min, run_min)
        new_min = new_min.astype(jnp.bfloat16).astype(jnp.float32)
        return (new_min, jnp.where(better, loc_arg, run_arg))

    init = (jnp.full((BB, 1), jnp.inf, jnp.float32),
            jnp.zeros((BB, 1), jnp.int32))
    _, arg = jax.lax.fori_loop(0, K // KC, body, init)
    out_ref[...] = arg


def _vq_shard(x, codebook):
    nb = x.shape[0]
    x_sq = jnp.sum(x * x, axis=-1, keepdims=True)
    c_sq = jnp.sum(codebook * codebook, axis=-1)[None, :]
    out = pl.pallas_call(
        _vq_kernel,
        grid=(nb // BB,),
        in_specs=[
            pl.BlockSpec((BB, D), lambda i: (i, 0)),
            pl.BlockSpec((K, D), lambda i: (0, 0)),
            pl.BlockSpec((BB, 1), lambda i: (i, 0)),
            pl.BlockSpec((1, K), lambda i: (0, 0)),
        ],
        out_specs=pl.BlockSpec((BB, 1), lambda i: (i, 0)),
        out_shape=jax.ShapeDtypeStruct((nb, 1), jnp.int32),
    )(x, codebook, x_sq, c_sq)
    return out.reshape(nb)


def kernel(x, codebook):
    devs = jax.devices()
    if len(devs) >= 2:
        mesh = Mesh(np.array(devs[:2]), ("d",))
        f = jax.shard_map(_vq_shard, mesh=mesh,
                          in_specs=(P("d", None), P(None, None)),
                          out_specs=P("d"), check_vma=False)
        return f(x, codebook)
    return _vq_shard(x, codebook)


# speedup vs baseline: 2.0021x; 2.0021x over previous
import jax
import jax.numpy as jnp
from jax.experimental import pallas as pl

B, K, D = 8192, 8192, 64
BB = 512
KC = 2048
LANES = 128
JJ = KC // LANES


def _vq_kernel(x2_ref, cb_ref, xsq_ref, csq_ref, out_ref):
    x2 = x2_ref[...]
    x_sq = xsq_ref[...]
    jvec = jax.lax.broadcasted_iota(
        jnp.int32, (BB, JJ, LANES), 1).astype(jnp.float32)
    lane = jax.lax.broadcasted_iota(
        jnp.int32, (BB, LANES), 1).astype(jnp.float32)

    def body(kc, carry):
        run_min, run_arg = carry
        c = cb_ref[pl.ds(kc * KC, KC), :]
        c_sq = csq_ref[:, pl.ds(kc * KC, KC)]
        cross2 = jax.lax.dot_general(
            x2, c, (((1,), (1,)), ((), ())),
            preferred_element_type=jnp.float32)
        d = (x_sq + c_sq) + cross2
        dr = d.reshape(BB, JJ, LANES)
        m_lane = jnp.min(dr, axis=1)
        jbest = jnp.min(jnp.where(dr == m_lane[:, None, :], jvec, 1e9),
                        axis=1)
        m_row = jnp.min(m_lane, axis=1, keepdims=True)
        k_lane = jbest * float(LANES) + lane
        cand = jnp.where(m_lane == m_row, k_lane, 1e9)
        k_row = jnp.min(cand, axis=1, keepdims=True)
        loc_arg = k_row.astype(jnp.int32) + kc * KC
        better = m_row < run_min
        new_min = jnp.where(better, m_row, run_min)
        new_min = new_min.astype(jnp.bfloat16).astype(jnp.float32)
        return (new_min, jnp.where(better, loc_arg, run_arg))

    init = (jnp.full((BB, 1), jnp.inf, jnp.float32),
            jnp.zeros((BB, 1), jnp.int32))
    _, arg = jax.lax.fori_loop(0, K // KC, body, init)
    out_ref[...] = arg


def _vq_single(x, codebook):
    x_sq = jnp.sum(x * x, axis=-1, keepdims=True)
    c_sq = jnp.sum(codebook * codebook, axis=-1)[None, :]
    x2 = x * (-2.0)
    out = pl.pallas_call(
        _vq_kernel,
        grid=(B // BB,),
        in_specs=[
            pl.BlockSpec((BB, D), lambda i: (i, 0)),
            pl.BlockSpec((K, D), lambda i: (0, 0)),
            pl.BlockSpec((BB, 1), lambda i: (i, 0)),
            pl.BlockSpec((1, K), lambda i: (0, 0)),
        ],
        out_specs=pl.BlockSpec((BB, 1), lambda i: (i, 0)),
        out_shape=jax.ShapeDtypeStruct((B, 1), jnp.int32),
    )(x2, codebook, x_sq, c_sq)
    return out.reshape(B)


def kernel(x, codebook):
    return _vq_single(x, codebook)


# speedup vs baseline: 3.7252x; 1.8606x over previous
import jax
import jax.numpy as jnp
from jax.experimental import pallas as pl

B, K, D = 8192, 8192, 64
BB = 512
KC = 2048
LANES = 128
JJ = KC // LANES


def _vq_kernel(x2_ref, cb_ref, xsq_ref, csq_ref, out_ref):
    x2 = x2_ref[...]
    x_sq = xsq_ref[...]
    kvec = jax.lax.broadcasted_iota(
        jnp.int32, (BB, KC), 1).astype(jnp.float32)

    def body(kc, carry):
        run_min, run_arg = carry
        c = cb_ref[pl.ds(kc * KC, KC), :]
        c_sq = csq_ref[:, pl.ds(kc * KC, KC)]
        cross2 = jax.lax.dot_general(
            x2, c, (((1,), (1,)), ((), ())),
            preferred_element_type=jnp.float32)
        d = (x_sq + c_sq) + cross2
        m_row = jnp.min(d, axis=1, keepdims=True)
        k_row = jnp.min(jnp.where(d == m_row, kvec, 1e9),
                        axis=1, keepdims=True)
        loc_arg = k_row.astype(jnp.int32) + kc * KC
        better = m_row < run_min
        new_min = jnp.where(better, m_row, run_min)
        new_min = new_min.astype(jnp.bfloat16).astype(jnp.float32)
        return (new_min, jnp.where(better, loc_arg, run_arg))

    init = (jnp.full((BB, 1), jnp.inf, jnp.float32),
            jnp.zeros((BB, 1), jnp.int32))
    _, arg = jax.lax.fori_loop(0, K // KC, body, init)
    out_ref[...] = arg


def _vq_single(x, codebook):
    x_sq = jnp.sum(x * x, axis=-1, keepdims=True)
    c_sq = jnp.sum(codebook * codebook, axis=-1)[None, :]
    x2 = x * (-2.0)
    out = pl.pallas_call(
        _vq_kernel,
        grid=(B // BB,),
        in_specs=[
            pl.BlockSpec((BB, D), lambda i: (i, 0)),
            pl.BlockSpec((K, D), lambda i: (0, 0)),
            pl.BlockSpec((BB, 1), lambda i: (i, 0)),
            pl.BlockSpec((1, K), lambda i: (0, 0)),
        ],
        out_specs=pl.BlockSpec((BB, 1), lambda i: (i, 0)),
        out_shape=jax.ShapeDtypeStruct((B, 1), jnp.int32),
    )(x2, codebook, x_sq, c_sq)
    return out.reshape(B)


def kernel(x, codebook):
    return _vq_single(x, codebook)


# speedup vs baseline: 4.4946x; 1.2065x over previous
import jax
import jax.numpy as jnp
from jax.experimental import pallas as pl

B, K, D = 8192, 8192, 64
BB = 512
KC = 2048
LANES = 128
JJ = KC // LANES


def _vq_kernel(x2_ref, cb_ref, xsq_ref, csq_ref, out_ref):
    x2 = x2_ref[...]
    x_sq = xsq_ref[...]
    kvec = jax.lax.broadcasted_iota(
        jnp.int32, (BB, KC), 1).astype(jnp.float32)

    run_min = jnp.full((BB, 1), jnp.inf, jnp.float32)
    run_arg = jnp.zeros((BB, 1), jnp.int32)
    for kc in range(K // KC):
        c = cb_ref[pl.ds(kc * KC, KC), :]
        c_sq = csq_ref[:, pl.ds(kc * KC, KC)]
        cross2 = jax.lax.dot_general(
            x2, c, (((1,), (1,)), ((), ())),
            preferred_element_type=jnp.float32)
        d = (x_sq + c_sq) + cross2
        m_row = jnp.min(d, axis=1, keepdims=True)
        k_row = jnp.min(jnp.where(d == m_row, kvec, 1e9),
                        axis=1, keepdims=True)
        loc_arg = k_row.astype(jnp.int32) + kc * KC
        better = m_row < run_min
        new_min = jnp.where(better, m_row, run_min)
        run_min = new_min.astype(jnp.bfloat16).astype(jnp.float32)
        run_arg = jnp.where(better, loc_arg, run_arg)
    out_ref[...] = run_arg


def _vq_single(x, codebook):
    x_sq = jnp.sum(x * x, axis=-1, keepdims=True)
    c_sq = jnp.sum(codebook * codebook, axis=-1)[None, :]
    x2 = x * (-2.0)
    out = pl.pallas_call(
        _vq_kernel,
        grid=(B // BB,),
        in_specs=[
            pl.BlockSpec((BB, D), lambda i: (i, 0)),
            pl.BlockSpec((K, D), lambda i: (0, 0)),
            pl.BlockSpec((BB, 1), lambda i: (i, 0)),
            pl.BlockSpec((1, K), lambda i: (0, 0)),
        ],
        out_specs=pl.BlockSpec((BB, 1), lambda i: (i, 0)),
        out_shape=jax.ShapeDtypeStruct((B, 1), jnp.int32),
    )(x2, codebook, x_sq, c_sq)
    return out.reshape(B)


def kernel(x, codebook):
    return _vq_single(x, codebook)


# speedup vs baseline: 4.5708x; 1.0169x over previous
import jax
import jax.numpy as jnp
from jax.experimental import pallas as pl

B, K, D = 8192, 8192, 64
BB = 1024
KC = 2048
LANES = 128
JJ = KC // LANES


def _vq_kernel(x2_ref, cb_ref, xsq_ref, csq_ref, out_ref):
    x2 = x2_ref[...]
    x_sq = xsq_ref[...]
    kvec = jax.lax.broadcasted_iota(
        jnp.int32, (BB, KC), 1).astype(jnp.float32)

    run_min = jnp.full((BB, 1), jnp.inf, jnp.float32)
    run_arg = jnp.zeros((BB, 1), jnp.int32)
    for kc in range(K // KC):
        c = cb_ref[pl.ds(kc * KC, KC), :]
        c_sq = csq_ref[:, pl.ds(kc * KC, KC)]
        cross2 = jax.lax.dot_general(
            x2, c, (((1,), (1,)), ((), ())),
            preferred_element_type=jnp.float32)
        d = (x_sq + c_sq) + cross2
        m_row = jnp.min(d, axis=1, keepdims=True)
        k_row = jnp.min(jnp.where(d == m_row, kvec, 1e9),
                        axis=1, keepdims=True)
        loc_arg = k_row.astype(jnp.int32) + kc * KC
        better = m_row < run_min
        new_min = jnp.where(better, m_row, run_min)
        run_min = new_min.astype(jnp.bfloat16).astype(jnp.float32)
        run_arg = jnp.where(better, loc_arg, run_arg)
    out_ref[...] = run_arg


def _vq_single(x, codebook):
    x_sq = jnp.sum(x * x, axis=-1, keepdims=True)
    c_sq = jnp.sum(codebook * codebook, axis=-1)[None, :]
    x2 = x * (-2.0)
    out = pl.pallas_call(
        _vq_kernel,
        grid=(B // BB,),
        in_specs=[
            pl.BlockSpec((BB, D), lambda i: (i, 0)),
            pl.BlockSpec((K, D), lambda i: (0, 0)),
            pl.BlockSpec((BB, 1), lambda i: (i, 0)),
            pl.BlockSpec((1, K), lambda i: (0, 0)),
        ],
        out_specs=pl.BlockSpec((BB, 1), lambda i: (i, 0)),
        out_shape=jax.ShapeDtypeStruct((B, 1), jnp.int32),
    )(x2, codebook, x_sq, c_sq)
    return out.reshape(B)


def kernel(x, codebook):
    return _vq_single(x, codebook)


# speedup vs baseline: 4.7657x; 1.0426x over previous
import jax
import jax.numpy as jnp
from jax.experimental import pallas as pl

B, K, D = 8192, 8192, 64
BB = 1024
KC = 2048
LANES = 128
JJ = KC // LANES


def _vq_kernel(x_ref, cb_ref, xsq_ref, csq_ref, out_ref):
    x2 = x_ref[...] * (-2.0)
    x_sq = xsq_ref[...]
    kvec = jax.lax.broadcasted_iota(
        jnp.int32, (BB, KC), 1).astype(jnp.float32)

    run_min = jnp.full((BB, 1), jnp.inf, jnp.float32)
    run_arg = jnp.zeros((BB, 1), jnp.int32)
    for kc in range(K // KC):
        c = cb_ref[pl.ds(kc * KC, KC), :]
        c_sq = csq_ref[:, pl.ds(kc * KC, KC)]
        cross2 = jax.lax.dot_general(
            x2, c, (((1,), (1,)), ((), ())),
            preferred_element_type=jnp.float32)
        d = (x_sq + c_sq) + cross2
        m_row = jnp.min(d, axis=1, keepdims=True)
        k_row = jnp.min(jnp.where(d == m_row, kvec, 1e9),
                        axis=1, keepdims=True)
        loc_arg = k_row.astype(jnp.int32) + kc * KC
        better = m_row < run_min
        new_min = jnp.where(better, m_row, run_min)
        run_min = new_min.astype(jnp.bfloat16).astype(jnp.float32)
        run_arg = jnp.where(better, loc_arg, run_arg)
    out_ref[...] = run_arg


def _vq_single(x, codebook):
    x_sq = jnp.sum(x * x, axis=-1, keepdims=True)
    c_sq = jnp.sum(codebook * codebook, axis=-1)[None, :]
    out = pl.pallas_call(
        _vq_kernel,
        grid=(B // BB,),
        in_specs=[
            pl.BlockSpec((BB, D), lambda i: (i, 0)),
            pl.BlockSpec((K, D), lambda i: (0, 0)),
            pl.BlockSpec((BB, 1), lambda i: (i, 0)),
            pl.BlockSpec((1, K), lambda i: (0, 0)),
        ],
        out_specs=pl.BlockSpec((BB, 1), lambda i: (i, 0)),
        out_shape=jax.ShapeDtypeStruct((B, 1), jnp.int32),
    )(x, codebook, x_sq, c_sq)
    return out.reshape(B)


def kernel(x, codebook):
    return _vq_single(x, codebook)


# speedup vs baseline: 4.7679x; 1.0005x over previous
import jax
import jax.numpy as jnp
from jax.experimental import pallas as pl

B, K, D = 8192, 8192, 64
BB = 1024
KC = 2048


def _vq_kernel(x_ref, cb_ref, xsq_ref, csq_ref, out_ref):
    x2 = x_ref[...] * (-2.0)
    x_sq = xsq_ref[...]
    kvec = jax.lax.broadcasted_iota(
        jnp.int32, (BB, KC), 1).astype(jnp.float32)

    run_min = jnp.full((BB, 1), jnp.inf, jnp.float32)
    run_arg = jnp.zeros((BB, 1), jnp.int32)
    for kc in range(K // KC):
        c = cb_ref[pl.ds(kc * KC, KC), :]
        c_sq = csq_ref[:, pl.ds(kc * KC, KC)]
        cross2 = jax.lax.dot_general(
            x2, c, (((1,), (1,)), ((), ())),
            preferred_element_type=jnp.float32)
        d = (x_sq + c_sq) + cross2
        m_row = jnp.min(d, axis=1, keepdims=True)
        k_row = jnp.min(jnp.where(d == m_row, kvec, 1e9),
                        axis=1, keepdims=True)
        loc_arg = k_row.astype(jnp.int32) + kc * KC
        better = m_row < run_min
        new_min = jnp.where(better, m_row, run_min)
        run_min = new_min.astype(jnp.bfloat16).astype(jnp.float32)
        run_arg = jnp.where(better, loc_arg, run_arg)
    out_ref[...] = run_arg


def kernel(x, codebook):
    x_sq = jnp.sum(x * x, axis=-1, keepdims=True)
    c_sq = jnp.sum(codebook * codebook, axis=-1)[None, :]
    out = pl.pallas_call(
        _vq_kernel,
        grid=(B // BB,),
        in_specs=[
            pl.BlockSpec((BB, D), lambda i: (i, 0)),
            pl.BlockSpec((K, D), lambda i: (0, 0)),
            pl.BlockSpec((BB, 1), lambda i: (i, 0)),
            pl.BlockSpec((1, K), lambda i: (0, 0)),
        ],
        out_specs=pl.BlockSpec((BB, 1), lambda i: (i, 0)),
        out_shape=jax.ShapeDtypeStruct((B, 1), jnp.int32),
    )(x, codebook, x_sq, c_sq)
    return out.reshape(B)
